# baseline (speedup 1.0000x reference)
import functools

import jax
import jax.numpy as jnp
from jax import lax
from jax.experimental import pallas as pl
from jax.experimental.pallas import tpu as pltpu
from jax.experimental.pallas import tpu_sc as plsc

B, L, C = 8, 512, 52
V, E, D = 128, 64, 256

NC, NS, LANES = 2, 16, 16
NW = NC * NS
NTOK = B * L
TPW = NTOK // NW
GRP = 2
RPG = GRP * C
NGRP = TPW // GRP
NCHUNK = D // LANES


def _prep_body(order_ref, slen_ref, ids_ref, idx_ref):
    m = pl.program_id(0)
    sl = slen_ref[m]
    ids = ids_ref[0]
    pos = jax.lax.broadcasted_iota(jnp.int32, (L, C), 0)
    coff = jax.lax.broadcasted_iota(jnp.int32, (L, C), 1) * V
    idx_ref[0] = jnp.where(pos < sl, ids, 0) + coff


def _prep(x_ids, order, slen):
    grid_spec = pltpu.PrefetchScalarGridSpec(
        num_scalar_prefetch=2,
        grid=(B,),
        in_specs=[
            pl.BlockSpec((1, L, C), lambda m, order_ref, slen_ref: (order_ref[m], 0, 0)),
        ],
        out_specs=pl.BlockSpec((1, L, C), lambda m, *_: (m, 0, 0)),
    )
    return pl.pallas_call(
        _prep_body,
        grid_spec=grid_spec,
        out_shape=jax.ShapeDtypeStruct((B, L, C), jnp.int32),
    )(order, slen, x_ids)


def _table_body(emb_ref, w_ref, b_ref, t_ref):
    c = pl.program_id(0)
    t = jax.lax.dot(emb_ref[...], w_ref[0], preferred_element_type=jnp.float32)
    bias = jnp.where(c == 0, b_ref[...], 0.0)
    t_ref[0] = t + bias


def _build_table(emb, w3, b2):
    return pl.pallas_call(
        _table_body,
        grid=(C,),
        in_specs=[
            pl.BlockSpec((V, E), lambda c: (0, 0)),
            pl.BlockSpec((1, E, D), lambda c: (c, 0, 0)),
            pl.BlockSpec((1, D), lambda c: (0, 0)),
        ],
        out_specs=pl.BlockSpec((1, V, D), lambda c: (c, 0, 0)),
        out_shape=jax.ShapeDtypeStruct((C, V, D), jnp.float32),
    )(emb, w3, b2)


def _sc_body(t_ref, idx_ref, out_ref, idx_v, rows_v, out_v, sem0, sem1):
    w = lax.axis_index("s") * NC + lax.axis_index("c")
    pltpu.sync_copy(idx_ref.at[w], idx_v)
    sems = (sem0, sem1)

    def start(gg, buf):
        pltpu.make_async_copy(t_ref.at[idx_v.at[gg]], rows_v.at[buf], sems[buf]).start()

    def process(gg, buf):
        pltpu.make_async_copy(t_ref.at[idx_v.at[gg]], rows_v.at[buf], sems[buf]).wait()
        for tok in range(GRP):
            base = tok * C

            def acc_body(r, carry):
                return tuple(
                    carry[v] + rows_v[buf, base + r, pl.ds(v * LANES, LANES)]
                    for v in range(NCHUNK)
                )

            acc = lax.fori_loop(
                0, C, acc_body,
                tuple(jnp.zeros((LANES,), jnp.float32) for _ in range(NCHUNK)),
            )
            for v in range(NCHUNK):
                out_v[gg * GRP + tok, pl.ds(v * LANES, LANES)] = jnp.maximum(acc[v], 0.0)

    start(0, 0)
    start(1, 1)

    def loop_body(g, _):
        for buf in range(2):
            process(g + buf, buf)
            start(g + buf + 2, buf)
        return 0

    lax.fori_loop(0, (NGRP - 2) // 2, lambda i, c: loop_body(2 * i, c), 0)
    process(NGRP - 2, 0)
    process(NGRP - 1, 1)
    pltpu.sync_copy(out_v, out_ref.at[pl.ds(w * TPW, TPW)])


def _sc_gather_sum(t_flat, idx3):
    mesh = plsc.VectorSubcoreMesh(core_axis_name="c", subcore_axis_name="s")
    f = functools.partial(
        pl.kernel,
        out_type=jax.ShapeDtypeStruct((NTOK, D), jnp.float32),
        mesh=mesh,
        scratch_types=[
            pltpu.VMEM((NGRP, RPG), jnp.int32),
            pltpu.VMEM((2, RPG, D), jnp.float32),
            pltpu.VMEM((TPW, D), jnp.float32),
            pltpu.SemaphoreType.DMA,
            pltpu.SemaphoreType.DMA,
        ],
    )(_sc_body)
    return f(t_flat, idx3)


def kernel(x_ids, lengths, emb, W, b):
    order = jnp.argsort(-lengths, stable=True).astype(jnp.int32)
    sorted_len = lengths[order]
    reversed_indices = jnp.argsort(order, stable=True)

    idx = _prep(x_ids.astype(jnp.int32), order, sorted_len.astype(jnp.int32))
    t = _build_table(emb, W.reshape(C, E, D), b.reshape(1, D))
    y = _sc_gather_sum(t.reshape(C * V, D), idx.reshape(NW, NGRP, RPG))
    return (y.reshape(B, L, D), sorted_len, reversed_indices)
